# baseline (speedup 1.0000x reference)
import functools

import jax
import jax.numpy as jnp
from jax import lax
from jax.experimental import pallas as pl
from jax.experimental.pallas import tpu as pltpu
from jax.experimental.pallas import tpu_sc as plsc

B, L, D = 4096, 200, 128
N = B * L
NC, NS = 2, 16
NW = NC * NS
TOK_PER_W = N // NW
K = 128
NCHUNK = TOK_PER_W // K
DW = D // 2
V_DT, V_WD, V_DY = 1441, 8, 367
V_DTP = 1536
DT_PER_TILE = V_DTP // NS
V_CB = 3072
CB_PER_TILE = V_CB // NS
CB_PASS = 48
LANES = 16
GROUPS = D // 32

_HI16 = -65536


def _to_bf16_bits(w):
    return w + 0x7FFF + (lax.shift_right_logical(w, 16) & 1)


def _pack_rows(src, dst, nrows):
    def row(i, _):
        for g in range(GROUPS):
            wl = lax.bitcast_convert_type(
                src[i, pl.ds(g * 32, LANES)], jnp.int32)
            wh = lax.bitcast_convert_type(
                src[i, pl.ds(g * 32 + LANES, LANES)], jnp.int32)
            lo16 = lax.shift_right_logical(_to_bf16_bits(wl), 16)
            hi16 = _to_bf16_bits(wh) & _HI16
            dst[i, pl.ds(g * LANES, LANES)] = lo16 | hi16
        return ()
    lax.fori_loop(0, nrows, row, (), unroll=4)


def _emb_body(dt_idx, wd_idx, dy_idx, dt_tab, wd_tab, dy_tab, out,
              dt_sp, cb_sp, i1, i2, i3, i23,
              ci_dy, ci_wd, tb, pb, rb1, rb2, r, sem_i, sem_g, sem_o):
    sid = lax.axis_index("s")
    wid = sid * NC + lax.axis_index("c")
    w_base = wid * TOK_PER_W

    for q in range(DT_PER_TILE // CB_PASS):
        dt0 = pl.multiple_of(sid * DT_PER_TILE + q * CB_PASS, CB_PASS)
        pltpu.sync_copy(dt_tab.at[pl.ds(dt0, CB_PASS)], tb)
        _pack_rows(tb, pb, CB_PASS)
        pltpu.sync_copy(pb, dt_sp.at[pl.ds(dt0, CB_PASS)])

    for p in range(CB_PER_TILE // CB_PASS):
        cb0 = pl.multiple_of(sid * CB_PER_TILE + p * CB_PASS, CB_PASS)
        for j in range(CB_PASS // LANES):
            v = cb0 + j * LANES + lax.iota(jnp.int32, 16)
            ci_dy[pl.ds(j * LANES, LANES)] = jnp.minimum(v >> 3, V_DY - 1)
            ci_wd[pl.ds(j * LANES, LANES)] = v & 7
        pltpu.async_copy(dy_tab.at[ci_dy], tb, sem_g.at[0]).wait()
        pltpu.async_copy(wd_tab.at[ci_wd], tb, sem_g.at[0], add=True).wait()
        _pack_rows(tb, pb, CB_PASS)
        pltpu.sync_copy(pb, cb_sp.at[pl.ds(cb0, CB_PASS)])
    plsc.subcore_barrier()

    def prefetch_idx(c, b):
        s = pl.ds(w_base + c * K, K)
        d = pl.ds(b * K, K)
        pltpu.async_copy(dt_idx.at[s], i1.at[d], sem_i.at[b])
        pltpu.async_copy(wd_idx.at[s], i2.at[d], sem_i.at[b])
        pltpu.async_copy(dy_idx.at[s], i3.at[d], sem_i.at[b])

    def wait_idx(b):
        d = pl.ds(b * K, K)
        pltpu.make_async_copy(dt_idx.at[pl.ds(0, K)], i1.at[d], sem_i.at[b]).wait()
        pltpu.make_async_copy(wd_idx.at[pl.ds(0, K)], i2.at[d], sem_i.at[b]).wait()
        pltpu.make_async_copy(dy_idx.at[pl.ds(0, K)], i3.at[d], sem_i.at[b]).wait()

    def wait_gathers(b):
        pltpu.make_async_copy(dt_sp.at[i1.at[pl.ds(b * K, K)]], rb1.at[b],
                              sem_g.at[b]).wait()
        pltpu.make_async_copy(cb_sp.at[i23.at[pl.ds(b * K, K)]], rb2.at[b],
                              sem_g.at[b]).wait()

    def decode_add_chunk(b):
        def row(i, _):
            for g in range(GROUPS):
                w1 = rb1[b, i, pl.ds(g * LANES, LANES)]
                w2 = rb2[b, i, pl.ds(g * LANES, LANES)]
                lo = (lax.bitcast_convert_type(lax.shift_left(w1, 16),
                                               jnp.float32)
                      + lax.bitcast_convert_type(lax.shift_left(w2, 16),
                                                 jnp.float32))
                hi = (lax.bitcast_convert_type(w1 & _HI16, jnp.float32)
                      + lax.bitcast_convert_type(w2 & _HI16, jnp.float32))
                r[b, i, pl.ds(g * 32, LANES)] = lo
                r[b, i, pl.ds(g * 32 + LANES, LANES)] = hi
            return ()
        lax.fori_loop(0, K, row, (), unroll=4)

    prefetch_idx(0, 0)
    prefetch_idx(1, 1)

    def chunk(c, b):
        wait_idx(b)
        for j in range(K // LANES):
            s16 = pl.ds(b * K + j * LANES, LANES)
            i23[s16] = (i3[s16] << 3) + i2[s16]
        pltpu.async_copy(dt_sp.at[i1.at[pl.ds(b * K, K)]], rb1.at[b],
                         sem_g.at[b])
        pltpu.async_copy(cb_sp.at[i23.at[pl.ds(b * K, K)]], rb2.at[b],
                         sem_g.at[b])

        @pl.when(c >= 3)
        def _():
            pltpu.make_async_copy(r.at[1 - b], out.at[pl.ds(w_base, K)],
                                  sem_o.at[1 - b]).wait()

        @pl.when(c >= 1)
        def _():
            wait_gathers(1 - b)
            decode_add_chunk(1 - b)
            pltpu.async_copy(
                r.at[1 - b],
                out.at[pl.ds(w_base + (c - 1) * K, K)], sem_o.at[1 - b])

        @pl.when(c + 2 < NCHUNK)
        def _():
            prefetch_idx(c + 2, b)

    def pair(p, _):
        chunk(2 * p, 0)
        chunk(2 * p + 1, 1)
        return ()

    lax.fori_loop(0, NCHUNK // 2, pair, ())

    b_last = (NCHUNK - 1) % 2
    pltpu.make_async_copy(r.at[b_last], out.at[pl.ds(w_base, K)],
                          sem_o.at[b_last]).wait()
    wait_gathers(b_last)
    decode_add_chunk(b_last)
    pltpu.sync_copy(r.at[b_last], out.at[pl.ds(w_base + (NCHUNK - 1) * K, K)])
    pltpu.make_async_copy(r.at[1 - b_last], out.at[pl.ds(w_base, K)],
                          sem_o.at[1 - b_last]).wait()


@functools.partial(jax.jit, static_argnames=())
def kernel(daytime, weekday, day, daytime_table, weekday_table, day_table):
    dt = daytime.reshape(N).astype(jnp.int32)
    wd = weekday.reshape(N).astype(jnp.int32)
    dy = day.reshape(N).astype(jnp.int32)
    dt_tab_p = jnp.pad(daytime_table, ((0, V_DTP - V_DT), (0, 0)))

    mesh = plsc.VectorSubcoreMesh(core_axis_name="c", subcore_axis_name="s")
    run = pl.kernel(
        _emb_body,
        out_type=jax.ShapeDtypeStruct((N, D), jnp.float32),
        mesh=mesh,
        compiler_params=pltpu.CompilerParams(use_tc_tiling_on_sc=False),
        scratch_types=[
            pltpu.VMEM_SHARED((V_DTP, DW), jnp.int32),
            pltpu.VMEM_SHARED((V_CB, DW), jnp.int32),
            pltpu.VMEM((2 * K,), jnp.int32),
            pltpu.VMEM((2 * K,), jnp.int32),
            pltpu.VMEM((2 * K,), jnp.int32),
            pltpu.VMEM((2 * K,), jnp.int32),
            pltpu.VMEM((CB_PASS,), jnp.int32),
            pltpu.VMEM((CB_PASS,), jnp.int32),
            pltpu.VMEM((CB_PASS, D), jnp.float32),
            pltpu.VMEM((CB_PASS, DW), jnp.int32),
            pltpu.VMEM((2, K, DW), jnp.int32),
            pltpu.VMEM((2, K, DW), jnp.int32),
            pltpu.VMEM((2, K, D), jnp.float32),
            pltpu.SemaphoreType.DMA((2,)),
            pltpu.SemaphoreType.DMA((2,)),
            pltpu.SemaphoreType.DMA((2,)),
        ],
    )
    out = run(dt, wd, dy, dt_tab_p, weekday_table, day_table)
    return out.reshape(B, L, D)


# speedup vs baseline: 2.3751x; 2.3751x over previous
import functools

import jax
import jax.numpy as jnp
from jax import lax
from jax.experimental import pallas as pl
from jax.experimental.pallas import tpu as pltpu
from jax.experimental.pallas import tpu_sc as plsc

B, L, D = 4096, 200, 128
N = B * L
NC, NS = 2, 16
NW = NC * NS
TOK_PER_W = N // NW
K = 128
NCHUNK = TOK_PER_W // K
DW = D // 2
V_DT, V_WD, V_DY = 1441, 8, 367
V_DTP = 1536
DT_PER_TILE = V_DTP // NS
V_CB = 3072
CB_PER_TILE = V_CB // NS
CB_PASS = 48
LANES = 16
GROUPS = D // 32

_HI16 = -65536


def _to_bf16_bits(w):
    return w + 0x7FFF + (lax.shift_right_logical(w, 16) & 1)


def _pack_rows(src, dst, nrows):
    def row(i, _):
        for g in range(GROUPS):
            wl = lax.bitcast_convert_type(
                src[i, pl.ds(g * 32, LANES)], jnp.int32)
            wh = lax.bitcast_convert_type(
                src[i, pl.ds(g * 32 + LANES, LANES)], jnp.int32)
            lo16 = lax.shift_right_logical(_to_bf16_bits(wl), 16)
            hi16 = _to_bf16_bits(wh) & _HI16
            dst[i, pl.ds(g * LANES, LANES)] = lo16 | hi16
        return ()
    lax.fori_loop(0, nrows, row, (), unroll=4)


def _emb_body(dt_idx, wd_idx, dy_idx, dt_tab, wd_tab, dy_tab, out,
              dt_sp, cb_sp, i1, i2, i3, i23,
              ci_dy, ci_wd, tb, pb, rb1, rb2, r, sem_i, sem_g, sem_o):
    sid = lax.axis_index("s")
    wid = sid * NC + lax.axis_index("c")
    w_base = wid * TOK_PER_W

    for q in range(DT_PER_TILE // CB_PASS):
        dt0 = pl.multiple_of(sid * DT_PER_TILE + q * CB_PASS, CB_PASS)
        pltpu.sync_copy(dt_tab.at[pl.ds(dt0, CB_PASS)], tb)
        _pack_rows(tb, pb, CB_PASS)
        pltpu.sync_copy(pb, dt_sp.at[pl.ds(dt0, CB_PASS)])

    for p in range(CB_PER_TILE // CB_PASS):
        cb0 = pl.multiple_of(sid * CB_PER_TILE + p * CB_PASS, CB_PASS)
        for j in range(CB_PASS // LANES):
            v = cb0 + j * LANES + lax.iota(jnp.int32, 16)
            ci_dy[pl.ds(j * LANES, LANES)] = jnp.minimum(v >> 3, V_DY - 1)
            ci_wd[pl.ds(j * LANES, LANES)] = v & 7
        pltpu.async_copy(dy_tab.at[ci_dy], tb, sem_g.at[0]).wait()
        pltpu.async_copy(wd_tab.at[ci_wd], tb, sem_g.at[0], add=True).wait()
        _pack_rows(tb, pb, CB_PASS)
        pltpu.sync_copy(pb, cb_sp.at[pl.ds(cb0, CB_PASS)])
    plsc.subcore_barrier()

    def prefetch_idx(c, b):
        s = pl.ds(w_base + c * K, K)
        d = pl.ds(b * K, K)
        pltpu.async_copy(dt_idx.at[s], i1.at[d], sem_i.at[b])
        pltpu.async_copy(wd_idx.at[s], i2.at[d], sem_i.at[b])
        pltpu.async_copy(dy_idx.at[s], i3.at[d], sem_i.at[b])

    def wait_idx(b):
        d = pl.ds(b * K, K)
        pltpu.make_async_copy(dt_idx.at[pl.ds(0, K)], i1.at[d], sem_i.at[b]).wait()
        pltpu.make_async_copy(wd_idx.at[pl.ds(0, K)], i2.at[d], sem_i.at[b]).wait()
        pltpu.make_async_copy(dy_idx.at[pl.ds(0, K)], i3.at[d], sem_i.at[b]).wait()

    def wait_gathers(b):
        pltpu.make_async_copy(dt_sp.at[i1.at[pl.ds(b * K, K)]], rb1.at[b],
                              sem_g.at[b]).wait()
        pltpu.make_async_copy(cb_sp.at[i23.at[pl.ds(b * K, K)]], rb2.at[b],
                              sem_g.at[b]).wait()

    def decode_add_chunk(b):
        @plsc.parallel_loop(0, K, unroll=8)
        def _(i):
            for g in range(GROUPS):
                w1 = rb1[b, i, pl.ds(g * LANES, LANES)]
                w2 = rb2[b, i, pl.ds(g * LANES, LANES)]
                lo = (lax.bitcast_convert_type(lax.shift_left(w1, 16),
                                               jnp.float32)
                      + lax.bitcast_convert_type(lax.shift_left(w2, 16),
                                                 jnp.float32))
                hi = (lax.bitcast_convert_type(w1 & _HI16, jnp.float32)
                      + lax.bitcast_convert_type(w2 & _HI16, jnp.float32))
                r[b, i, pl.ds(g * 32, LANES)] = lo
                r[b, i, pl.ds(g * 32 + LANES, LANES)] = hi

    prefetch_idx(0, 0)
    prefetch_idx(1, 1)

    def chunk(c, b):
        wait_idx(b)
        for j in range(K // LANES):
            s16 = pl.ds(b * K + j * LANES, LANES)
            i23[s16] = (i3[s16] << 3) + i2[s16]
        pltpu.async_copy(dt_sp.at[i1.at[pl.ds(b * K, K)]], rb1.at[b],
                         sem_g.at[b])
        pltpu.async_copy(cb_sp.at[i23.at[pl.ds(b * K, K)]], rb2.at[b],
                         sem_g.at[b])

        @pl.when(c >= 3)
        def _():
            pltpu.make_async_copy(r.at[1 - b], out.at[pl.ds(w_base, K)],
                                  sem_o.at[1 - b]).wait()

        @pl.when(c >= 1)
        def _():
            wait_gathers(1 - b)
            decode_add_chunk(1 - b)
            pltpu.async_copy(
                r.at[1 - b],
                out.at[pl.ds(w_base + (c - 1) * K, K)], sem_o.at[1 - b])

        @pl.when(c + 2 < NCHUNK)
        def _():
            prefetch_idx(c + 2, b)

    def pair(p, _):
        chunk(2 * p, 0)
        chunk(2 * p + 1, 1)
        return ()

    lax.fori_loop(0, NCHUNK // 2, pair, ())

    b_last = (NCHUNK - 1) % 2
    pltpu.make_async_copy(r.at[b_last], out.at[pl.ds(w_base, K)],
                          sem_o.at[b_last]).wait()
    wait_gathers(b_last)
    decode_add_chunk(b_last)
    pltpu.sync_copy(r.at[b_last], out.at[pl.ds(w_base + (NCHUNK - 1) * K, K)])
    pltpu.make_async_copy(r.at[1 - b_last], out.at[pl.ds(w_base, K)],
                          sem_o.at[1 - b_last]).wait()


@functools.partial(jax.jit, static_argnames=())
def kernel(daytime, weekday, day, daytime_table, weekday_table, day_table):
    dt = daytime.reshape(N).astype(jnp.int32)
    wd = weekday.reshape(N).astype(jnp.int32)
    dy = day.reshape(N).astype(jnp.int32)
    dt_tab_p = jnp.pad(daytime_table, ((0, V_DTP - V_DT), (0, 0)))

    mesh = plsc.VectorSubcoreMesh(core_axis_name="c", subcore_axis_name="s")
    run = pl.kernel(
        _emb_body,
        out_type=jax.ShapeDtypeStruct((N, D), jnp.float32),
        mesh=mesh,
        compiler_params=pltpu.CompilerParams(use_tc_tiling_on_sc=False),
        scratch_types=[
            pltpu.VMEM_SHARED((V_DTP, DW), jnp.int32),
            pltpu.VMEM_SHARED((V_CB, DW), jnp.int32),
            pltpu.VMEM((2 * K,), jnp.int32),
            pltpu.VMEM((2 * K,), jnp.int32),
            pltpu.VMEM((2 * K,), jnp.int32),
            pltpu.VMEM((2 * K,), jnp.int32),
            pltpu.VMEM((CB_PASS,), jnp.int32),
            pltpu.VMEM((CB_PASS,), jnp.int32),
            pltpu.VMEM((CB_PASS, D), jnp.float32),
            pltpu.VMEM((CB_PASS, DW), jnp.int32),
            pltpu.VMEM((2, K, DW), jnp.int32),
            pltpu.VMEM((2, K, DW), jnp.int32),
            pltpu.VMEM((2, K, D), jnp.float32),
            pltpu.SemaphoreType.DMA((2,)),
            pltpu.SemaphoreType.DMA((2,)),
            pltpu.SemaphoreType.DMA((2,)),
        ],
    )
    out = run(dt, wd, dy, dt_tab_p, weekday_table, day_table)
    return out.reshape(B, L, D)
